# baseline (speedup 1.0000x reference)
import functools

import jax
import jax.numpy as jnp
from jax.experimental import pallas as pl
from jax.experimental.pallas import tpu as pltpu
from jax.experimental.pallas import tpu_sc as plsc

_N = 5000
_C = 21
_MH = 14
_MW = 14
_PAD = 300
_ROWS = 40
_LANES = 128
_NP = _ROWS * _LANES
_NW = 32
_BPW = 16
_GPAD = _NW * _BPW
_D = _MH * _MW * _C
_DP = 4224
_SCPAD = _N + 8
_ZROW = _N + 4
_BIG = 2**30


def _nms_body(meta_ref, x1_ref, y1_ref, x2_ref, y2_ref, sc_ref, dl_ref,
              pb_ref, idx_ref, cnt_ref):
    scale = meta_ref[0, 2]
    h_img = meta_ref[0, 0]
    w_img = meta_ref[0, 1]
    x1 = x1_ref[...] / scale
    y1 = y1_ref[...] / scale
    x2 = x2_ref[...] / scale
    y2 = y2_ref[...] / scale
    wa = x2 - x1 + 1.0
    ha = y2 - y1 + 1.0
    cxa = x1 + 0.5 * wa
    cya = y1 + 0.5 * ha

    best = sc_ref[0]
    top = jnp.zeros((_ROWS, _LANES), jnp.int32)
    maxsc = sc_ref[1]
    for c in range(1, _C):
        plane = sc_ref[c]
        top = jnp.where(plane > best, c, top)
        best = jnp.maximum(best, plane)
        if c > 1:
            maxsc = jnp.maximum(maxsc, plane)

    dx = dl_ref[0]
    dy = dl_ref[1]
    dw = dl_ref[2]
    dh = dl_ref[3]
    for c in range(1, _C):
        sel = top == c
        dx = jnp.where(sel, dl_ref[4 * c + 0], dx)
        dy = jnp.where(sel, dl_ref[4 * c + 1], dy)
        dw = jnp.where(sel, dl_ref[4 * c + 2], dw)
        dh = jnp.where(sel, dl_ref[4 * c + 3], dh)

    cx = dx * wa + cxa
    cy = dy * ha + cya
    w = jnp.exp(dw) * wa
    h = jnp.exp(dh) * ha
    px1 = jnp.clip(cx - 0.5 * w, 0.0, w_img - 1.0)
    py1 = jnp.clip(cy - 0.5 * h, 0.0, h_img - 1.0)
    px2 = jnp.clip(cx + 0.5 * w, 0.0, w_img - 1.0)
    py2 = jnp.clip(cy + 0.5 * h, 0.0, h_img - 1.0)
    areas = (px2 - px1 + 1.0) * (py2 - py1 + 1.0)

    row = jax.lax.broadcasted_iota(jnp.int32, (_ROWS, _LANES), 0)
    lane = jax.lax.broadcasted_iota(jnp.int32, (_ROWS, _LANES), 1)
    fidx = row * _LANES + lane
    in_bounds = fidx < _N
    pb_ref[0] = jnp.where(in_bounds, px1, 0.0)
    pb_ref[1] = jnp.where(in_bounds, py1, 0.0)
    pb_ref[2] = jnp.where(in_bounds, px2, 0.0)
    pb_ref[3] = jnp.where(in_bounds, py2, 0.0)

    cand0 = jnp.where(in_bounds, 1.0, 0.0).astype(jnp.float32)
    neg_inf = jnp.float32(-jnp.inf)

    def _red(op, x):
        return op(op(x, axis=1, keepdims=True), axis=0, keepdims=True)

    def body(k, state):
        candf, cnt = state
        cand = candf > 0.0
        masked = jnp.where(cand, maxsc, neg_inf)
        m_val = _red(jnp.max, masked)
        is_m = (masked == m_val) & cand
        m_idx = _red(jnp.min, jnp.where(is_m, fidx, _BIG))
        sel = fidx == m_idx
        bx1 = _red(jnp.sum, jnp.where(sel, px1, 0.0))
        by1 = _red(jnp.sum, jnp.where(sel, py1, 0.0))
        bx2 = _red(jnp.sum, jnp.where(sel, px2, 0.0))
        by2 = _red(jnp.sum, jnp.where(sel, py2, 0.0))
        ba = _red(jnp.sum, jnp.where(sel, areas, 0.0))
        iw = jnp.maximum(jnp.minimum(bx2, px2) - jnp.maximum(bx1, px1) + 1.0,
                         0.0)
        ih = jnp.maximum(jnp.minimum(by2, py2) - jnp.maximum(by1, py1) + 1.0,
                         0.0)
        inter = iw * ih
        iou = inter / (ba + areas - inter)
        newcandf = jnp.where(cand & ~((iou > 0.5) | sel), 1.0, 0.0).astype(
            jnp.float32)
        m_scalar = m_idx[0, 0]
        alive = m_scalar < _BIG
        idx_ref[k] = jnp.where(alive, m_scalar, 0)
        return newcandf, cnt + jnp.where(alive, 1, 0).astype(jnp.int32)

    _, kfin = jax.lax.fori_loop(0, _PAD, body, (cand0, jnp.int32(0)))
    cnt_ref[0] = kfin

    def ztail(i, carry):
        idx_ref[i] = 0
        return carry

    jax.lax.fori_loop(_PAD, _GPAD, ztail, 0)


def _sc_gather(idx_hbm, cnt_hbm, pb_hbm, masks_hbm, sc_hbm, zeros_hbm,
               masks_out, sc_out, pb_out,
               idx_v, cnt_v, masks_v, sc_v, pb_v, sem):
    wid = jax.lax.axis_index("s") * 2 + jax.lax.axis_index("c")
    base = wid * _BPW
    pltpu.sync_copy(idx_hbm.at[pl.ds(base, _BPW)], idx_v)
    pltpu.sync_copy(cnt_hbm, cnt_v)
    idx = idx_v[...]
    cnt_vec = cnt_v[...]
    j16 = jax.lax.broadcasted_iota(jnp.int32, (_BPW,), 0)
    invalid = (j16 + base) >= cnt_vec
    idx_z = jnp.where(invalid, _ZROW, idx)
    cp = pltpu.async_copy(masks_hbm.at[idx], masks_v, sem)
    pltpu.sync_copy(sc_hbm.at[idx_z], sc_v)
    pltpu.sync_copy(pb_hbm.at[idx_z], pb_v)
    cp.wait()
    c = cnt_v[...][0]
    for j in range(_BPW):
        @pl.when(base + j >= c)
        def _():
            pltpu.sync_copy(zeros_hbm, masks_v.at[j])
    pltpu.sync_copy(masks_v, masks_out.at[pl.ds(base, _BPW)])
    pltpu.sync_copy(sc_v, sc_out.at[pl.ds(base, _BPW)])
    pltpu.sync_copy(pb_v, pb_out.at[pl.ds(base, _BPW)])


@jax.jit
def kernel(metadata, deltas, proposals, scores, masks):
    p = proposals.reshape(_N, 4)
    pad = _NP - _N
    planes = [jnp.pad(p[:, k], (0, pad)).reshape(_ROWS, _LANES) for k in range(4)]
    sc = scores.reshape(_N, _C)
    sct = jnp.pad(sc.T, ((0, 0), (0, pad))).reshape(_C, _ROWS, _LANES)
    dlt = jnp.pad(deltas.reshape(_N, 4 * _C).T, ((0, 0), (0, pad))).reshape(
        4 * _C, _ROWS, _LANES)

    pb, idx, cnt = pl.pallas_call(
        _nms_body,
        out_shape=[
            jax.ShapeDtypeStruct((4, _ROWS, _LANES), jnp.float32),
            jax.ShapeDtypeStruct((_GPAD,), jnp.int32),
            jax.ShapeDtypeStruct((1,), jnp.int32),
        ],
        in_specs=[
            pl.BlockSpec(memory_space=pltpu.SMEM),
            pl.BlockSpec(memory_space=pltpu.VMEM),
            pl.BlockSpec(memory_space=pltpu.VMEM),
            pl.BlockSpec(memory_space=pltpu.VMEM),
            pl.BlockSpec(memory_space=pltpu.VMEM),
            pl.BlockSpec(memory_space=pltpu.VMEM),
            pl.BlockSpec(memory_space=pltpu.VMEM),
        ],
        out_specs=[
            pl.BlockSpec(memory_space=pltpu.VMEM),
            pl.BlockSpec(memory_space=pltpu.SMEM),
            pl.BlockSpec(memory_space=pltpu.SMEM),
        ],
    )(metadata, *planes, sct, dlt)

    pb_tab = jnp.pad(pb.reshape(4, _NP).T, ((0, 0), (0, 124)))
    masks_tab = jnp.pad(masks.reshape(_N, _D), ((0, 0), (0, _DP - _D)))
    sc_tab = jnp.pad(sc, ((0, _SCPAD - _N), (0, 128 - _C)))
    zeros_row = jnp.zeros((_DP,), jnp.float32)
    cnt16 = jnp.broadcast_to(cnt, (_BPW,)).astype(jnp.int32)

    sc_fn = pl.kernel(
        _sc_gather,
        out_type=[
            jax.ShapeDtypeStruct((_GPAD, _DP), jnp.float32),
            jax.ShapeDtypeStruct((_GPAD, 128), jnp.float32),
            jax.ShapeDtypeStruct((_GPAD, 128), jnp.float32),
        ],
        mesh=plsc.VectorSubcoreMesh(core_axis_name="c", subcore_axis_name="s"),
        scratch_types=[
            pltpu.VMEM((_BPW,), jnp.int32),
            pltpu.VMEM((_BPW,), jnp.int32),
            pltpu.VMEM((_BPW, _DP), jnp.float32),
            pltpu.VMEM((_BPW, 128), jnp.float32),
            pltpu.VMEM((_BPW, 128), jnp.float32),
            pltpu.SemaphoreType.DMA,
        ],
    )
    masks_out, sc_out, pb_out = sc_fn(idx, cnt16, pb_tab, masks_tab, sc_tab,
                                      zeros_row)

    out_boxes = pb_out[:_PAD, :4][None]
    out_scores = sc_out[:_PAD, :_C][None]
    out_masks = masks_out[:_PAD, :_D].reshape(_PAD, _MH, _MW, _C)[None]
    return out_boxes, out_scores, out_masks


# speedup vs baseline: 2.6277x; 2.6277x over previous
import functools

import jax
import jax.numpy as jnp
from jax.experimental import pallas as pl
from jax.experimental.pallas import tpu as pltpu
from jax.experimental.pallas import tpu_sc as plsc

_N = 5000
_C = 21
_MH = 14
_MW = 14
_PAD = 300
_ROWS = 40
_LANES = 128
_NP = _ROWS * _LANES
_NW = 32
_BPW = 16
_GPAD = _NW * _BPW
_D = _MH * _MW * _C
_DP = 4224
_SCPAD = _N + 8
_ZROW = _N + 4
_BIG = 2**30


def _nms_body(meta_ref, x1_ref, y1_ref, x2_ref, y2_ref, sc_ref, dl_ref,
              pb_ref, idx_ref, cnt_ref):
    scale = meta_ref[0, 2]
    h_img = meta_ref[0, 0]
    w_img = meta_ref[0, 1]
    x1 = x1_ref[...] / scale
    y1 = y1_ref[...] / scale
    x2 = x2_ref[...] / scale
    y2 = y2_ref[...] / scale
    wa = x2 - x1 + 1.0
    ha = y2 - y1 + 1.0
    cxa = x1 + 0.5 * wa
    cya = y1 + 0.5 * ha

    best = sc_ref[0]
    top = jnp.zeros((_ROWS, _LANES), jnp.int32)
    maxsc = sc_ref[1]
    for c in range(1, _C):
        plane = sc_ref[c]
        top = jnp.where(plane > best, c, top)
        best = jnp.maximum(best, plane)
        if c > 1:
            maxsc = jnp.maximum(maxsc, plane)

    dx = dl_ref[0]
    dy = dl_ref[1]
    dw = dl_ref[2]
    dh = dl_ref[3]
    for c in range(1, _C):
        sel = top == c
        dx = jnp.where(sel, dl_ref[4 * c + 0], dx)
        dy = jnp.where(sel, dl_ref[4 * c + 1], dy)
        dw = jnp.where(sel, dl_ref[4 * c + 2], dw)
        dh = jnp.where(sel, dl_ref[4 * c + 3], dh)

    cx = dx * wa + cxa
    cy = dy * ha + cya
    w = jnp.exp(dw) * wa
    h = jnp.exp(dh) * ha
    px1 = jnp.clip(cx - 0.5 * w, 0.0, w_img - 1.0)
    py1 = jnp.clip(cy - 0.5 * h, 0.0, h_img - 1.0)
    px2 = jnp.clip(cx + 0.5 * w, 0.0, w_img - 1.0)
    py2 = jnp.clip(cy + 0.5 * h, 0.0, h_img - 1.0)
    areas = (px2 - px1 + 1.0) * (py2 - py1 + 1.0)

    row = jax.lax.broadcasted_iota(jnp.int32, (_ROWS, _LANES), 0)
    lane = jax.lax.broadcasted_iota(jnp.int32, (_ROWS, _LANES), 1)
    fidx = row * _LANES + lane
    in_bounds = fidx < _N
    pb_ref[0] = jnp.where(in_bounds, px1, 0.0)
    pb_ref[1] = jnp.where(in_bounds, py1, 0.0)
    pb_ref[2] = jnp.where(in_bounds, px2, 0.0)
    pb_ref[3] = jnp.where(in_bounds, py2, 0.0)

    cand0 = jnp.where(in_bounds, 1.0, 0.0).astype(jnp.float32)
    neg_inf = jnp.float32(-jnp.inf)

    def _red(op, x):
        return op(op(x, axis=1, keepdims=True), axis=0, keepdims=True)

    def body(k, state):
        candf, cnt = state
        cand = candf > 0.0
        masked = jnp.where(cand, maxsc, neg_inf)
        m_val = _red(jnp.max, masked)
        is_m = (masked == m_val) & cand
        m_idx = _red(jnp.min, jnp.where(is_m, fidx, _BIG))
        sel = fidx == m_idx
        bx1 = _red(jnp.sum, jnp.where(sel, px1, 0.0))
        by1 = _red(jnp.sum, jnp.where(sel, py1, 0.0))
        bx2 = _red(jnp.sum, jnp.where(sel, px2, 0.0))
        by2 = _red(jnp.sum, jnp.where(sel, py2, 0.0))
        ba = _red(jnp.sum, jnp.where(sel, areas, 0.0))
        iw = jnp.maximum(jnp.minimum(bx2, px2) - jnp.maximum(bx1, px1) + 1.0,
                         0.0)
        ih = jnp.maximum(jnp.minimum(by2, py2) - jnp.maximum(by1, py1) + 1.0,
                         0.0)
        inter = iw * ih
        iou = inter / (ba + areas - inter)
        newcandf = jnp.where(cand & ~((iou > 0.5) | sel), 1.0, 0.0).astype(
            jnp.float32)
        m_scalar = m_idx[0, 0]
        alive = m_scalar < _BIG
        idx_ref[k] = jnp.where(alive, m_scalar, 0)
        return newcandf, cnt + jnp.where(alive, 1, 0).astype(jnp.int32)

    _, kfin = jax.lax.fori_loop(0, _PAD, body, (cand0, jnp.int32(0)))
    cnt_ref[0] = kfin

    def ztail(i, carry):
        idx_ref[i] = 0
        return carry

    jax.lax.fori_loop(_PAD, _GPAD, ztail, 0)


def _sc_gather(idx_hbm, cnt_hbm, pb_hbm, sc_hbm,
               sc_out, pb_out,
               idx_v, cnt_v, sc_v, pb_v):
    wid = jax.lax.axis_index("s") * 2 + jax.lax.axis_index("c")
    base = wid * _BPW
    pltpu.sync_copy(idx_hbm.at[pl.ds(base, _BPW)], idx_v)
    pltpu.sync_copy(cnt_hbm, cnt_v)
    idx = idx_v[...]
    cnt_vec = cnt_v[...]
    j16 = jax.lax.broadcasted_iota(jnp.int32, (_BPW,), 0)
    invalid = (j16 + base) >= cnt_vec
    idx_z = jnp.where(invalid, _ZROW, idx)
    pltpu.sync_copy(sc_hbm.at[idx_z], sc_v)
    pltpu.sync_copy(pb_hbm.at[idx_z], pb_v)
    pltpu.sync_copy(sc_v, sc_out.at[pl.ds(base, _BPW)])
    pltpu.sync_copy(pb_v, pb_out.at[pl.ds(base, _BPW)])


def _mask_gather_body(idx_ref, cnt_ref, masks_ref, masks_out_ref):
    i = pl.program_id(0)
    v = jnp.where(i < cnt_ref[0], jnp.float32(1.0), jnp.float32(0.0))
    masks_out_ref[...] = masks_ref[...] * v


@jax.jit
def kernel(metadata, deltas, proposals, scores, masks):
    p = proposals.reshape(_N, 4)
    pad = _NP - _N
    planes = [jnp.pad(p[:, k], (0, pad)).reshape(_ROWS, _LANES) for k in range(4)]
    sc = scores.reshape(_N, _C)
    sct = jnp.pad(sc.T, ((0, 0), (0, pad))).reshape(_C, _ROWS, _LANES)
    dlt = jnp.pad(deltas.reshape(_N, 4 * _C).T, ((0, 0), (0, pad))).reshape(
        4 * _C, _ROWS, _LANES)

    pb, idx, cnt = pl.pallas_call(
        _nms_body,
        out_shape=[
            jax.ShapeDtypeStruct((4, _ROWS, _LANES), jnp.float32),
            jax.ShapeDtypeStruct((_GPAD,), jnp.int32),
            jax.ShapeDtypeStruct((1,), jnp.int32),
        ],
        in_specs=[
            pl.BlockSpec(memory_space=pltpu.SMEM),
            pl.BlockSpec(memory_space=pltpu.VMEM),
            pl.BlockSpec(memory_space=pltpu.VMEM),
            pl.BlockSpec(memory_space=pltpu.VMEM),
            pl.BlockSpec(memory_space=pltpu.VMEM),
            pl.BlockSpec(memory_space=pltpu.VMEM),
            pl.BlockSpec(memory_space=pltpu.VMEM),
        ],
        out_specs=[
            pl.BlockSpec(memory_space=pltpu.VMEM),
            pl.BlockSpec(memory_space=pltpu.SMEM),
            pl.BlockSpec(memory_space=pltpu.SMEM),
        ],
    )(metadata, *planes, sct, dlt)

    pb_tab = jnp.pad(pb.reshape(4, _NP).T, ((0, 0), (0, 124)))
    sc_tab = jnp.pad(sc, ((0, _SCPAD - _N), (0, 128 - _C)))
    cnt16 = jnp.broadcast_to(cnt, (_BPW,)).astype(jnp.int32)

    sc_fn = pl.kernel(
        _sc_gather,
        out_type=[
            jax.ShapeDtypeStruct((_GPAD, 128), jnp.float32),
            jax.ShapeDtypeStruct((_GPAD, 128), jnp.float32),
        ],
        mesh=plsc.VectorSubcoreMesh(core_axis_name="c", subcore_axis_name="s"),
        scratch_types=[
            pltpu.VMEM((_BPW,), jnp.int32),
            pltpu.VMEM((_BPW,), jnp.int32),
            pltpu.VMEM((_BPW, 128), jnp.float32),
            pltpu.VMEM((_BPW, 128), jnp.float32),
        ],
    )
    sc_out, pb_out = sc_fn(idx, cnt16, pb_tab, sc_tab)

    grid_spec = pltpu.PrefetchScalarGridSpec(
        num_scalar_prefetch=2,
        grid=(_PAD,),
        in_specs=[
            pl.BlockSpec((1, _MH, _MW, _C), lambda i, idx, cnt: (idx[i], 0, 0, 0)),
        ],
        out_specs=pl.BlockSpec((1, _MH, _MW, _C),
                               lambda i, idx, cnt: (i, 0, 0, 0)),
    )
    masks_out = pl.pallas_call(
        _mask_gather_body,
        grid_spec=grid_spec,
        out_shape=jax.ShapeDtypeStruct((_PAD, _MH, _MW, _C), jnp.float32),
    )(idx[:_PAD], cnt, masks[0])

    out_boxes = pb_out[:_PAD, :4][None]
    out_scores = sc_out[:_PAD, :_C][None]
    out_masks = masks_out[None]
    return out_boxes, out_scores, out_masks


# speedup vs baseline: 4.0633x; 1.5463x over previous
import functools

import jax
import jax.numpy as jnp
from jax.experimental import pallas as pl
from jax.experimental.pallas import tpu as pltpu
from jax.experimental.pallas import tpu_sc as plsc

_N = 5000
_C = 21
_MH = 14
_MW = 14
_PAD = 300
_ROWS = 40
_LANES = 128
_NP = _ROWS * _LANES
_NW = 32
_BPW = 16
_GPAD = _NW * _BPW
_D = _MH * _MW * _C
_DP = 4224
_SCPAD = _N + 8
_ZROW = _N + 4
_BIG = 2**30


def _nms_body(meta_ref, x1_ref, y1_ref, x2_ref, y2_ref, sc_ref, dl_ref,
              pb_ref, idx_ref, cnt_ref):
    scale = meta_ref[0, 2]
    h_img = meta_ref[0, 0]
    w_img = meta_ref[0, 1]
    x1 = x1_ref[...] / scale
    y1 = y1_ref[...] / scale
    x2 = x2_ref[...] / scale
    y2 = y2_ref[...] / scale
    wa = x2 - x1 + 1.0
    ha = y2 - y1 + 1.0
    cxa = x1 + 0.5 * wa
    cya = y1 + 0.5 * ha

    best = sc_ref[0]
    top = jnp.zeros((_ROWS, _LANES), jnp.int32)
    maxsc = sc_ref[1]
    for c in range(1, _C):
        plane = sc_ref[c]
        top = jnp.where(plane > best, c, top)
        best = jnp.maximum(best, plane)
        if c > 1:
            maxsc = jnp.maximum(maxsc, plane)

    dx = dl_ref[0]
    dy = dl_ref[1]
    dw = dl_ref[2]
    dh = dl_ref[3]
    for c in range(1, _C):
        sel = top == c
        dx = jnp.where(sel, dl_ref[4 * c + 0], dx)
        dy = jnp.where(sel, dl_ref[4 * c + 1], dy)
        dw = jnp.where(sel, dl_ref[4 * c + 2], dw)
        dh = jnp.where(sel, dl_ref[4 * c + 3], dh)

    cx = dx * wa + cxa
    cy = dy * ha + cya
    w = jnp.exp(dw) * wa
    h = jnp.exp(dh) * ha
    px1 = jnp.clip(cx - 0.5 * w, 0.0, w_img - 1.0)
    py1 = jnp.clip(cy - 0.5 * h, 0.0, h_img - 1.0)
    px2 = jnp.clip(cx + 0.5 * w, 0.0, w_img - 1.0)
    py2 = jnp.clip(cy + 0.5 * h, 0.0, h_img - 1.0)
    areas = (px2 - px1 + 1.0) * (py2 - py1 + 1.0)

    row = jax.lax.broadcasted_iota(jnp.int32, (_ROWS, _LANES), 0)
    lane = jax.lax.broadcasted_iota(jnp.int32, (_ROWS, _LANES), 1)
    fidx = row * _LANES + lane
    in_bounds = fidx < _N
    pb_ref[0] = jnp.where(in_bounds, px1, 0.0)
    pb_ref[1] = jnp.where(in_bounds, py1, 0.0)
    pb_ref[2] = jnp.where(in_bounds, px2, 0.0)
    pb_ref[3] = jnp.where(in_bounds, py2, 0.0)

    cand0 = jnp.where(in_bounds, 1.0, 0.0).astype(jnp.float32)
    neg_inf = jnp.float32(-jnp.inf)

    def _red(op, x):
        return op(op(x, axis=1, keepdims=True), axis=0, keepdims=True)

    def body(k, state):
        candf, cnt = state
        cand = candf > 0.0
        masked = jnp.where(cand, maxsc, neg_inf)
        m_val = _red(jnp.max, masked)
        is_m = (masked == m_val) & cand
        m_idx = _red(jnp.min, jnp.where(is_m, fidx, _BIG))
        sel = fidx == m_idx
        bx1 = _red(jnp.sum, jnp.where(sel, px1, 0.0))
        by1 = _red(jnp.sum, jnp.where(sel, py1, 0.0))
        bx2 = _red(jnp.sum, jnp.where(sel, px2, 0.0))
        by2 = _red(jnp.sum, jnp.where(sel, py2, 0.0))
        ba = _red(jnp.sum, jnp.where(sel, areas, 0.0))
        iw = jnp.maximum(jnp.minimum(bx2, px2) - jnp.maximum(bx1, px1) + 1.0,
                         0.0)
        ih = jnp.maximum(jnp.minimum(by2, py2) - jnp.maximum(by1, py1) + 1.0,
                         0.0)
        inter = iw * ih
        iou = inter / (ba + areas - inter)
        newcandf = jnp.where(cand & ~((iou > 0.5) | sel), 1.0, 0.0).astype(
            jnp.float32)
        m_scalar = m_idx[0, 0]
        alive = m_scalar < _BIG
        idx_ref[k] = jnp.where(alive, m_scalar, 0)
        return newcandf, cnt + jnp.where(alive, 1, 0).astype(jnp.int32)

    _, kfin = jax.lax.fori_loop(0, _PAD, body, (cand0, jnp.int32(0)))
    cnt_ref[0] = kfin

    def ztail(i, carry):
        idx_ref[i] = 0
        return carry

    jax.lax.fori_loop(_PAD, _GPAD, ztail, 0)


def _sc_gather(idx_hbm, cnt_hbm, pb_hbm, sc_hbm,
               sc_out, pb_out,
               idx_v, cnt_v, sc_v, pb_v):
    wid = jax.lax.axis_index("s") * 2 + jax.lax.axis_index("c")
    base = wid * _BPW
    pltpu.sync_copy(idx_hbm.at[pl.ds(base, _BPW)], idx_v)
    pltpu.sync_copy(cnt_hbm, cnt_v)
    idx = idx_v[...]
    cnt_vec = cnt_v[...]
    j16 = jax.lax.broadcasted_iota(jnp.int32, (_BPW,), 0)
    invalid = (j16 + base) >= cnt_vec
    idx_z = jnp.where(invalid, _ZROW, idx)
    pltpu.sync_copy(sc_hbm.at[idx_z], sc_v)
    pltpu.sync_copy(pb_hbm.at[idx_z], pb_v)
    pltpu.sync_copy(sc_v, sc_out.at[pl.ds(base, _BPW)])
    pltpu.sync_copy(pb_v, pb_out.at[pl.ds(base, _BPW)])


_MROWS = 20
_MGRID = _PAD // _MROWS


def _mask_gather_body(idx_ref, cnt_ref, *refs):
    i = pl.program_id(0)
    cnt = cnt_ref[0]
    ins = refs[:_MROWS]
    out = refs[_MROWS]
    for j in range(_MROWS):
        v = jnp.where(i * _MROWS + j < cnt, jnp.float32(1.0), jnp.float32(0.0))
        out[pl.ds(j, 1)] = ins[j][...] * v


@jax.jit
def kernel(metadata, deltas, proposals, scores, masks):
    p = proposals.reshape(_N, 4)
    pad = _NP - _N
    planes = [jnp.pad(p[:, k], (0, pad)).reshape(_ROWS, _LANES) for k in range(4)]
    sc = scores.reshape(_N, _C)
    sct = jnp.pad(sc.T, ((0, 0), (0, pad))).reshape(_C, _ROWS, _LANES)
    dlt = jnp.pad(deltas.reshape(_N, 4 * _C).T, ((0, 0), (0, pad))).reshape(
        4 * _C, _ROWS, _LANES)

    pb, idx, cnt = pl.pallas_call(
        _nms_body,
        out_shape=[
            jax.ShapeDtypeStruct((4, _ROWS, _LANES), jnp.float32),
            jax.ShapeDtypeStruct((_GPAD,), jnp.int32),
            jax.ShapeDtypeStruct((1,), jnp.int32),
        ],
        in_specs=[
            pl.BlockSpec(memory_space=pltpu.SMEM),
            pl.BlockSpec(memory_space=pltpu.VMEM),
            pl.BlockSpec(memory_space=pltpu.VMEM),
            pl.BlockSpec(memory_space=pltpu.VMEM),
            pl.BlockSpec(memory_space=pltpu.VMEM),
            pl.BlockSpec(memory_space=pltpu.VMEM),
            pl.BlockSpec(memory_space=pltpu.VMEM),
        ],
        out_specs=[
            pl.BlockSpec(memory_space=pltpu.VMEM),
            pl.BlockSpec(memory_space=pltpu.SMEM),
            pl.BlockSpec(memory_space=pltpu.SMEM),
        ],
    )(metadata, *planes, sct, dlt)

    pb_tab = jnp.pad(pb.reshape(4, _NP).T, ((0, 0), (0, 124)))
    sc_tab = jnp.pad(sc, ((0, _SCPAD - _N), (0, 128 - _C)))
    cnt16 = jnp.broadcast_to(cnt, (_BPW,)).astype(jnp.int32)

    sc_fn = pl.kernel(
        _sc_gather,
        out_type=[
            jax.ShapeDtypeStruct((_GPAD, 128), jnp.float32),
            jax.ShapeDtypeStruct((_GPAD, 128), jnp.float32),
        ],
        mesh=plsc.VectorSubcoreMesh(core_axis_name="c", subcore_axis_name="s"),
        scratch_types=[
            pltpu.VMEM((_BPW,), jnp.int32),
            pltpu.VMEM((_BPW,), jnp.int32),
            pltpu.VMEM((_BPW, 128), jnp.float32),
            pltpu.VMEM((_BPW, 128), jnp.float32),
        ],
    )
    sc_out, pb_out = sc_fn(idx, cnt16, pb_tab, sc_tab)

    def _in_map(j):
        return lambda i, idx, cnt: (idx[i * _MROWS + j], 0, 0, 0)

    grid_spec = pltpu.PrefetchScalarGridSpec(
        num_scalar_prefetch=2,
        grid=(_MGRID,),
        in_specs=[pl.BlockSpec((1, _MH, _MW, _C), _in_map(j))
                  for j in range(_MROWS)],
        out_specs=pl.BlockSpec((_MROWS, _MH, _MW, _C),
                               lambda i, idx, cnt: (i, 0, 0, 0)),
    )
    masks_out = pl.pallas_call(
        _mask_gather_body,
        grid_spec=grid_spec,
        out_shape=jax.ShapeDtypeStruct((_PAD, _MH, _MW, _C), jnp.float32),
    )(idx[:_PAD], cnt, *([masks[0]] * _MROWS))

    out_boxes = pb_out[:_PAD, :4][None]
    out_scores = sc_out[:_PAD, :_C][None]
    out_masks = masks_out[None]
    return out_boxes, out_scores, out_masks
